# Initial kernel scaffold; baseline (speedup 1.0000x reference)
import jax
import jax.numpy as jnp
from jax.experimental import pallas as pl


def kernel(x, W_flat, b, flip_vals, flip_idx):
    raise NotImplementedError("write your pallas kernel here")


# speedup vs baseline: 4.5611x; 4.5611x over previous
import functools

import jax
import jax.numpy as jnp
from jax import lax
from jax.experimental import pallas as pl
from jax.experimental.pallas import tpu as pltpu
from jax.experimental.pallas import tpu_sc as plsc

D_IN = 4096
D_OUT = 4096
NUMEL = D_OUT * D_IN

NC = 2
NS = 16
NW = NC * NS
SB = 128
DEPTH = 8


def _copy_body(w_in, w_out):
    w_out[...] = w_in[...]


@functools.partial(jax.jit, static_argnums=())
def _tc_copy(w2d):
    blk = 256
    return pl.pallas_call(
        _copy_body,
        grid=(D_OUT // blk,),
        in_specs=[pl.BlockSpec((blk, D_IN), lambda i: (i, 0))],
        out_specs=pl.BlockSpec((blk, D_IN), lambda i: (i, 0)),
        out_shape=jax.ShapeDtypeStruct((D_OUT, D_IN), jnp.float32),
    )(w2d)


def _make_sc_scatter(K):
    mesh = plsc.VectorSubcoreMesh(
        core_axis_name="c", subcore_axis_name="s", num_cores=NC, num_subcores=NS
    )

    @functools.partial(
        pl.kernel,
        mesh=mesh,
        out_type=(),
        scratch_types=[
            pltpu.VMEM((K, SB), jnp.int32),
            pltpu.VMEM((K, SB), jnp.float32),
            pltpu.SemaphoreType.DMA,
        ],
    )
    def sc_scatter(w_hbm, idx_hbm, vals_hbm, idx_v, vals_v, sem):
        wid = lax.axis_index("s") * NC + lax.axis_index("c")
        pltpu.sync_copy(idx_hbm.at[wid], idx_v)
        pltpu.sync_copy(vals_hbm.at[wid], vals_v)

        def body(j, carry):
            @pl.when(j < K)
            def _fire():
                pltpu.async_copy(vals_v.at[j], w_hbm.at[idx_v.at[j]], sem)

            @pl.when(j >= DEPTH)
            def _drain():
                pltpu.make_async_copy(
                    vals_v.at[j - DEPTH], w_hbm.at[idx_v.at[j - DEPTH]], sem
                ).wait()

            return carry

        lax.fori_loop(0, K + DEPTH, body, 0)

    return sc_scatter


def _mm_body(x_ref, w_ref, b_ref, o_ref):
    acc = lax.dot_general(
        x_ref[...],
        w_ref[...],
        dimension_numbers=(((1,), (1,)), ((), ())),
        preferred_element_type=jnp.float32,
    )
    o_ref[...] = acc + b_ref[...][None, :]


def _tc_matmul(x, w2d, b):
    bn = 512
    batch = x.shape[0]
    return pl.pallas_call(
        _mm_body,
        grid=(D_OUT // bn,),
        in_specs=[
            pl.BlockSpec((batch, D_IN), lambda i: (0, 0)),
            pl.BlockSpec((bn, D_IN), lambda i: (i, 0)),
            pl.BlockSpec((bn,), lambda i: (i,)),
        ],
        out_specs=pl.BlockSpec((batch, bn), lambda i: (0, i)),
        out_shape=jax.ShapeDtypeStruct((batch, D_OUT), jnp.float32),
    )(x, w2d, b)


def kernel(x, W_flat, b, flip_vals, flip_idx):
    n = flip_idx.shape[0]
    chunk = NW * SB
    K = -(-n // chunk)
    npad = K * chunk - n

    idx = flip_idx.astype(jnp.int32)
    vals = flip_vals.astype(jnp.float32)
    if npad:
        idx = jnp.concatenate([idx, jnp.broadcast_to(idx[0], (npad,))])
        vals = jnp.concatenate([vals, jnp.broadcast_to(vals[0], (npad,))])
    idx3 = idx.reshape(NW, K, SB)
    vals3 = vals.reshape(NW, K, SB)

    w_mod = _tc_copy(W_flat.reshape(D_OUT, D_IN))
    wref = jax.new_ref(w_mod.reshape(NUMEL))
    _make_sc_scatter(K)(wref, idx3, vals3)
    w_final = jax.freeze(wref)

    return _tc_matmul(x, w_final.reshape(D_OUT, D_IN), b)
